# baseline (speedup 1.0000x reference)
import functools

import jax
import jax.numpy as jnp
from jax import lax
from jax.experimental import pallas as pl
from jax.experimental.pallas import tpu as pltpu
from jax.experimental.pallas import tpu_sc as plsc

VOCAB = 100000
EMB_DIM = 64
BATCH = 16384

_info = plsc.get_sparse_core_info()
_NC = _info.num_cores
_NS = _info.num_subcores
_NW = _NC * _NS
_BPW = BATCH // _NW
_NBLK = 4
_BLK = _BPW // _NBLK

_mesh = plsc.VectorSubcoreMesh(core_axis_name="c", subcore_axis_name="s")


@functools.partial(
    pl.kernel,
    mesh=_mesh,
    compiler_params=pltpu.CompilerParams(use_tc_tiling_on_sc=True),
    out_type=jax.ShapeDtypeStruct((BATCH, EMB_DIM), jnp.float32),
    scratch_types=[
        pltpu.VMEM((_BPW,), jnp.int32),
        pltpu.VMEM((_BPW, EMB_DIM), jnp.float32),
        pltpu.SemaphoreType.DMA,
        pltpu.SemaphoreType.DMA,
        pltpu.SemaphoreType.DMA,
        pltpu.SemaphoreType.DMA,
        pltpu.SemaphoreType.DMA,
    ],
)
def _gather_kernel(idx_hbm, table_hbm, out_hbm, idx_v, rows_v,
                   sem0, sem1, sem2, sem3, sem_o):
    wid = lax.axis_index("s") * _NC + lax.axis_index("c")
    base = wid * _BPW
    pltpu.sync_copy(idx_hbm.at[pl.ds(base, _BPW)], idx_v)

    sems = [sem0, sem1, sem2, sem3]
    for b in range(_NBLK):
        def fire(g, carry, b=b):
            v16 = idx_v[pl.ds(b * _BLK + g * 16, 16)]
            for j in range(16):
                pltpu.async_copy(
                    table_hbm.at[pl.ds(v16[j], 1)],
                    rows_v.at[pl.ds(b * _BLK + g * 16 + j, 1)],
                    sems[b],
                )
            return carry

        lax.fori_loop(0, _BLK // 16, fire, 0)

    outs = []
    for b in range(_NBLK):
        pltpu.make_async_copy(
            table_hbm.at[pl.ds(0, _BLK)],
            rows_v.at[pl.ds(b * _BLK, _BLK)],
            sems[b],
        ).wait()
        outs.append(pltpu.async_copy(
            rows_v.at[pl.ds(b * _BLK, _BLK)],
            out_hbm.at[pl.ds(base + b * _BLK, _BLK)],
            sem_o,
        ))
    for o in outs:
        o.wait()


def kernel(token_ids, embedding_weight):
    return _gather_kernel(token_ids.astype(jnp.int32), embedding_weight)


# speedup vs baseline: 1.0154x; 1.0154x over previous
import functools

import jax
import jax.numpy as jnp
from jax import lax
from jax.experimental import pallas as pl
from jax.experimental.pallas import tpu as pltpu
from jax.experimental.pallas import tpu_sc as plsc

VOCAB = 100000
EMB_DIM = 64
BATCH = 16384

_info = plsc.get_sparse_core_info()
_NC = _info.num_cores
_NS = _info.num_subcores
_NW = _NC * _NS
_BPW = BATCH // _NW

_mesh = plsc.VectorSubcoreMesh(core_axis_name="c", subcore_axis_name="s")


@functools.partial(
    pl.kernel,
    mesh=_mesh,
    compiler_params=pltpu.CompilerParams(use_tc_tiling_on_sc=True),
    out_type=jax.ShapeDtypeStruct((BATCH, EMB_DIM), jnp.float32),
    scratch_types=[
        pltpu.VMEM((_BPW,), jnp.int32),
        pltpu.VMEM((_BPW, EMB_DIM), jnp.float32),
        pltpu.SemaphoreType.DMA,
    ],
)
def _gather_kernel(idx_hbm, table_hbm, out_hbm, idx_v, rows_v, sem_g):
    wid = lax.axis_index("s") * _NC + lax.axis_index("c")
    base = wid * _BPW
    pltpu.sync_copy(idx_hbm.at[pl.ds(base, _BPW)], idx_v)

    def fire(g, carry):
        v16 = idx_v[pl.ds(g * 16, 16)]
        for j in range(16):
            pltpu.async_copy(
                table_hbm.at[pl.ds(v16[j], 1)],
                rows_v.at[pl.ds(g * 16 + j, 1)],
                sem_g,
            )
        return carry

    lax.fori_loop(0, _BPW // 16, fire, 0)
    pltpu.make_async_copy(
        table_hbm.at[pl.ds(0, _BPW)],
        rows_v,
        sem_g,
    ).wait()
    pltpu.sync_copy(rows_v, out_hbm.at[pl.ds(base, _BPW)])


def kernel(token_ids, embedding_weight):
    return _gather_kernel(token_ids.astype(jnp.int32), embedding_weight)
